# baseline (speedup 1.0000x reference)
import functools

import jax
import jax.numpy as jnp
from jax import lax
from jax.experimental import pallas as pl
from jax.experimental.pallas import tpu as pltpu
from jax.experimental.pallas import tpu_sc as plsc

NE = 50000
NU = 20000
C = 128
QW = 32
LO, HI = 42033, 44630
RD = HI - LO
RPAD = 2688
NEDGE = 600000
NNZ = 500000
SUP = 8
EP = 16 * 38 * SUP * 128
IP = 16 * 32 * SUP * 128
ER = EP // (SUP * 128)
IR = IP // (SUP * 128)
ET = 38
IT = 32
ES = 3128
US = 1256
ACC_E = 50008
ACC_U = 20008
DUM_E = NE
DUM_U = NU


def _region_blend(rwm_pad, ent_pad):
    def body(a_ref, b_ref, s_ref, o_ref):
        mm = jnp.dot(a_ref[...], b_ref[...],
                     preferred_element_type=jnp.float32)
        o_ref[...] = 0.8 * s_ref[...] + 0.2 * mm

    return pl.pallas_call(
        body,
        grid=(RPAD // 128,),
        in_specs=[
            pl.BlockSpec((128, RPAD), lambda i: (i, 0)),
            pl.BlockSpec((RPAD, 128), lambda i: (0, 0)),
            pl.BlockSpec((128, 128), lambda i: (i, 0)),
        ],
        out_specs=pl.BlockSpec((128, 128), lambda i: (i, 0)),
        out_shape=jax.ShapeDtypeStruct((RPAD, 128), jnp.float32),
    )(rwm_pad, ent_pad, ent_pad)


def _norm_ent(aggq, cnt, res_in):
    R = 400

    def body(a_ref, c_ref, r_ref, ro_ref, eo_ref):
        a = a_ref[...]
        x = jnp.concatenate([a[0], a[1], a[2], a[3]], axis=-1)
        c = jnp.maximum(c_ref[...][:, 0:1], 1.0)
        x = x / c
        n = jnp.sqrt(jnp.sum(x * x, axis=-1, keepdims=True))
        y = x / jnp.maximum(n, 1e-12)
        ro_ref[...] = r_ref[...] + y
        eo_ref[...] = y

    return pl.pallas_call(
        body,
        grid=(NE // R,),
        in_specs=[
            pl.BlockSpec((4, R, 32), lambda i: (0, i, 0)),
            pl.BlockSpec((R, 32), lambda i: (i, 0)),
            pl.BlockSpec((R, 128), lambda i: (i, 0)),
        ],
        out_specs=[pl.BlockSpec((R, 128), lambda i: (i, 0))] * 2,
        out_shape=[jax.ShapeDtypeStruct((NE, 128), jnp.float32)] * 2,
    )(aggq, cnt, res_in)


def _norm_user(aggq, res_in):
    R = 400

    def body(a_ref, r_ref, ro_ref):
        a = a_ref[...]
        x = jnp.concatenate([a[0], a[1], a[2], a[3]], axis=-1)
        n = jnp.sqrt(jnp.sum(x * x, axis=-1, keepdims=True))
        y = x / jnp.maximum(n, 1e-12)
        ro_ref[...] = r_ref[...] + y

    return pl.pallas_call(
        body,
        grid=(NU // R,),
        in_specs=[
            pl.BlockSpec((4, R, 32), lambda i: (0, i, 0)),
            pl.BlockSpec((R, 128), lambda i: (i, 0)),
        ],
        out_specs=pl.BlockSpec((R, 128), lambda i: (i, 0)),
        out_shape=jax.ShapeDtypeStruct((NU, 128), jnp.float32),
    )(aggq, res_in)


def _agg_pass(is_kg, q, sid, eqq, gsrc, ssrc, aux, wflat, zeros32, out, acc,
              tbuf, hbuf, ybuf, vbuf, X0, X1, Y0, Y1, wv,
              gx0, gx1, gy0, gy1, ssx, ssy, isem):
    nsup = ET if is_kg else IT
    tot, stripe = (ACC_E, ES) if is_kg else (ACC_U, US)
    off = pl.multiple_of(jnp.minimum(sid * stripe, tot - stripe), 8)
    if is_kg:
        pltpu.sync_copy(wflat.at[q], wv)
        pltpu.sync_copy(zeros32, acc.at[pl.ds(off, stripe)])
    else:
        pltpu.sync_copy(zeros32.at[pl.ds(0, US)], acc.at[pl.ds(off, US)])
    plsc.subcore_barrier()

    def mul(buf, row):
        if is_kg:
            @plsc.parallel_loop(0, 128, step=16, unroll=2)
            def _m(i):
                tv = ybuf[row, pl.ds(i, 16)] - 1
                for l in range(16):
                    woff = tv[l] * QW
                    w0 = wv[pl.ds(woff, 16)]
                    w1 = wv[pl.ds(woff + 16, 16)]
                    buf[i + l, pl.ds(0, 16)] = buf[i + l, pl.ds(0, 16)] * w0
                    buf[i + l, pl.ds(16, 16)] = buf[i + l, pl.ds(16, 16)] * w1
        else:
            @plsc.parallel_loop(0, 128, step=16, unroll=2)
            def _m(i):
                vv = vbuf[row, pl.ds(i, 16)]
                for l in range(16):
                    v = vv[l]
                    buf[i + l, pl.ds(0, 16)] = buf[i + l, pl.ds(0, 16)] * v
                    buf[i + l, pl.ds(16, 16)] = buf[i + l, pl.ds(16, 16)] * v

    abuf = ybuf if is_kg else vbuf
    sets = ((X0, X1, gx0, gx1, ssx), (Y0, Y1, gy0, gy1, ssy))
    base = sid * nsup

    def drain2(buf, sem):
        pltpu.make_async_copy(buf, acc.at[hbuf.at[0]], sem).wait()
        pltpu.make_async_copy(buf, acc.at[hbuf.at[0]], sem).wait()

    pltpu.sync_copy(gsrc.at[base], tbuf.at[pl.ds(0, SUP)])
    pltpu.sync_copy(ssrc.at[base], hbuf.at[pl.ds(0, SUP)])
    pltpu.sync_copy(aux.at[base], abuf.at[pl.ds(0, SUP)])
    pltpu.async_copy(eqq.at[tbuf.at[0]], X0, gx0)
    pltpu.async_copy(eqq.at[tbuf.at[1]], X1, gx1)

    def super_body(s, carry):
        paroff = (s % 2) * SUP
        nparoff = SUP - paroff
        for p in range(SUP // 2):
            cur0, cur1, g0, g1, scur = sets[p % 2]
            oth0, oth1, og0, og1, soth = sets[1 - p % 2]
            k0 = paroff + 2 * p
            k1 = k0 + 1
            pltpu.make_async_copy(eqq.at[tbuf.at[k0]], cur0, g0).wait()
            mul(cur0, k0)
            pltpu.make_async_copy(eqq.at[tbuf.at[k1]], cur1, g1).wait()
            mul(cur1, k1)
            pltpu.async_copy(cur0, acc.at[hbuf.at[k0]], scur, add=True)
            pltpu.async_copy(cur1, acc.at[hbuf.at[k1]], scur, add=True)
            if p == 0:
                @pl.when(s > 0)
                def _():
                    drain2(oth0, soth)

                @pl.when(s < nsup - 1)
                def _():
                    pltpu.async_copy(gsrc.at[base + s + 1],
                                     tbuf.at[pl.ds(nparoff, SUP)], isem)
                    pltpu.async_copy(ssrc.at[base + s + 1],
                                     hbuf.at[pl.ds(nparoff, SUP)], isem)
                    pltpu.async_copy(aux.at[base + s + 1],
                                     abuf.at[pl.ds(nparoff, SUP)], isem)
            else:
                drain2(oth0, soth)
            if p < SUP // 2 - 1:
                nk0 = k0 + 2
                pltpu.async_copy(eqq.at[tbuf.at[nk0]], oth0, og0)
                pltpu.async_copy(eqq.at[tbuf.at[nk0 + 1]], oth1, og1)
            else:
                @pl.when(s < nsup - 1)
                def _():
                    for aref in (tbuf, hbuf, abuf):
                        pltpu.make_async_copy(
                            gsrc.at[base], aref.at[pl.ds(nparoff, SUP)],
                            isem).wait()
                    pltpu.async_copy(eqq.at[tbuf.at[nparoff]], oth0, og0)
                    pltpu.async_copy(eqq.at[tbuf.at[nparoff + 1]], oth1, og1)
        return carry

    lax.fori_loop(0, nsup, super_body, 0)
    drain2(sets[(SUP // 2 - 1) % 2][0], sets[(SUP // 2 - 1) % 2][4])
    plsc.subcore_barrier()
    pltpu.sync_copy(acc.at[pl.ds(off, stripe)],
                    out.at[q].at[pl.ds(off, stripe)])
    plsc.subcore_barrier()


def _counts_pass(sid, ssrc, ones32, zeros32, oc, acc, hbuf, X0):
    off = pl.multiple_of(jnp.minimum(sid * ES, ACC_E - ES), 8)
    pltpu.sync_copy(zeros32, acc.at[pl.ds(off, ES)])
    plsc.subcore_barrier()
    pltpu.sync_copy(ones32, X0)
    base = sid * ET

    def super_body(s, carry):
        pltpu.sync_copy(ssrc.at[base + s], hbuf.at[pl.ds(0, SUP)])
        for k in range(SUP):
            pltpu.sync_copy(X0, acc.at[hbuf.at[k]], add=True)
        return carry

    lax.fori_loop(0, ET, super_body, 0)
    plsc.subcore_barrier()
    pltpu.sync_copy(acc.at[pl.ds(off, ES)], oc.at[pl.ds(off, ES)])
    plsc.subcore_barrier()


def _sc_body(do_counts, eq, heads, tails, types, irows, icols, ivals,
             wflat, ones32, zeros32, oe, ou, oc,
             acc, tbuf, hbuf, ybuf, vbuf, X0, X1, Y0, Y1, wv,
             gx0, gx1, gy0, gy1, ssx, ssy, isem):
    cid = lax.axis_index("c")
    sid = lax.axis_index("s")

    if do_counts:
        @pl.when(cid == 0)
        def _():
            _counts_pass(sid, heads, ones32, zeros32, oc, acc, hbuf, X0)

    def q_body(qi, carry):
        q = cid * 2 + qi
        eqq = eq.at[q]
        _agg_pass(True, q, sid, eqq, tails, heads, types, wflat, zeros32,
                  oe, acc, tbuf, hbuf, ybuf, vbuf, X0, X1, Y0, Y1, wv,
                  gx0, gx1, gy0, gy1, ssx, ssy, isem)
        _agg_pass(False, q, sid, eqq, icols, irows, ivals, wflat, zeros32,
                  ou, acc, tbuf, hbuf, ybuf, vbuf, X0, X1, Y0, Y1, wv,
                  gx0, gx1, gy0, gy1, ssx, ssy, isem)
        return carry

    lax.fori_loop(0, 2, q_body, 0)


@functools.lru_cache(maxsize=2)
def _sc_agg(do_counts):
    mesh = plsc.VectorSubcoreMesh(core_axis_name="c", subcore_axis_name="s",
                                  num_cores=2, num_subcores=16)
    out_type = [
        jax.ShapeDtypeStruct((4, ACC_E, 32), jnp.float32),
        jax.ShapeDtypeStruct((4, ACC_U, 32), jnp.float32),
        jax.ShapeDtypeStruct((ACC_E, 32), jnp.float32),
    ]
    scratch = [
        pltpu.VMEM_SHARED((ACC_E, 32), jnp.float32),
        pltpu.VMEM((2 * SUP, 128), jnp.int32),
        pltpu.VMEM((2 * SUP, 128), jnp.int32),
        pltpu.VMEM((2 * SUP, 128), jnp.int32),
        pltpu.VMEM((2 * SUP, 128), jnp.float32),
        pltpu.VMEM((128, 32), jnp.float32),
        pltpu.VMEM((128, 32), jnp.float32),
        pltpu.VMEM((128, 32), jnp.float32),
        pltpu.VMEM((128, 32), jnp.float32),
        pltpu.VMEM((384,), jnp.float32),
        pltpu.SemaphoreType.DMA,
        pltpu.SemaphoreType.DMA,
        pltpu.SemaphoreType.DMA,
        pltpu.SemaphoreType.DMA,
        pltpu.SemaphoreType.DMA,
        pltpu.SemaphoreType.DMA,
        pltpu.SemaphoreType.DMA,
    ]
    return pl.kernel(
        functools.partial(_sc_body, do_counts),
        out_type=out_type,
        mesh=mesh,
        scratch_types=scratch,
        compiler_params=pltpu.CompilerParams(use_tc_tiling_on_sc=False),
    )


def kernel(user_emb, entity_emb, edge_index, edge_type, interact_rows,
           interact_cols, interact_values, region_weight_mat, weight):
    head = edge_index[0]
    tail = edge_index[1]
    pe = EP - NEDGE
    heads_p = jnp.concatenate(
        [head, jnp.full((pe,), DUM_E, jnp.int32)]).reshape(ER, SUP, 128)
    tails_p = jnp.concatenate(
        [tail, jnp.zeros((pe,), jnp.int32)]).reshape(ER, SUP, 128)
    types_p = jnp.concatenate(
        [edge_type, jnp.ones((pe,), jnp.int32)]).reshape(ER, SUP, 128)
    pi = IP - NNZ
    irows_p = jnp.concatenate(
        [interact_rows,
         jnp.full((pi,), DUM_U, jnp.int32)]).reshape(IR, SUP, 128)
    icols_p = jnp.concatenate(
        [interact_cols, jnp.zeros((pi,), jnp.int32)]).reshape(IR, SUP, 128)
    ivals_p = jnp.concatenate(
        [interact_values,
         jnp.zeros((pi,), jnp.float32)]).reshape(IR, SUP, 128)
    wflat = jnp.pad(
        weight.reshape(11, 4, 32).transpose(1, 0, 2).reshape(4, 352),
        ((0, 0), (0, 32)))
    ones32 = jnp.ones((128, 32), jnp.float32)
    zeros32 = jnp.zeros((ES, 32), jnp.float32)
    rwm_pad = jnp.pad(region_weight_mat, ((0, RPAD - RD), (0, RPAD - RD)))

    ent = entity_emb
    e_res = entity_emb
    u_res = user_emb
    cnt = None
    for hop in range(2):
        ent_pad = jnp.pad(ent[LO:HI], ((0, RPAD - RD), (0, 0)))
        e_region = _region_blend(rwm_pad, ent_pad)[:RD]
        e = jnp.concatenate([ent[:LO], e_region, ent[HI:]], axis=0)
        eq = e.reshape(NE, 4, 32).transpose(1, 0, 2)
        oe, ou, oc = _sc_agg(hop == 0)(
            eq, heads_p, tails_p, types_p,
            irows_p, icols_p, ivals_p, wflat, ones32, zeros32)
        if hop == 0:
            cnt = oc[:NE]
        e_res, ent = _norm_ent(oe[:, :NE], cnt, e_res)
        u_res = _norm_user(ou[:, :NU], u_res)
    return e_res, u_res


# speedup vs baseline: 1.1879x; 1.1879x over previous
import functools

import jax
import jax.numpy as jnp
from jax import lax
from jax.experimental import pallas as pl
from jax.experimental.pallas import tpu as pltpu
from jax.experimental.pallas import tpu_sc as plsc

NE = 50000
NU = 20000
C = 128
QW = 32
LO, HI = 42033, 44630
RD = HI - LO
RPAD = 2688
NEDGE = 600000
NNZ = 500000
SUP = 8
PAR = 8
EP = 16 * 38 * SUP * 128
IP = 16 * 32 * SUP * 128
ER = EP // (SUP * 128)
IR = IP // (SUP * 128)
ET = 38
IT = 32
ES = 3128
US = 1256
ACC_E = 50008
ACC_U = 20008
DUM_E = NE
DUM_U = NU


def _region_blend(rwm_pad, ent_pad):
    def body(a_ref, b_ref, s_ref, o_ref):
        mm = jnp.dot(a_ref[...], b_ref[...],
                     preferred_element_type=jnp.float32)
        o_ref[...] = 0.8 * s_ref[...] + 0.2 * mm

    return pl.pallas_call(
        body,
        grid=(RPAD // 128,),
        in_specs=[
            pl.BlockSpec((128, RPAD), lambda i: (i, 0)),
            pl.BlockSpec((RPAD, 128), lambda i: (0, 0)),
            pl.BlockSpec((128, 128), lambda i: (i, 0)),
        ],
        out_specs=pl.BlockSpec((128, 128), lambda i: (i, 0)),
        out_shape=jax.ShapeDtypeStruct((RPAD, 128), jnp.float32),
    )(rwm_pad, ent_pad, ent_pad)


def _norm_ent(aggq, cnt, res_in):
    R = 400

    def body(a_ref, c_ref, r_ref, ro_ref, eo_ref):
        a = a_ref[...]
        x = jnp.concatenate([a[0], a[1], a[2], a[3]], axis=-1)
        c = jnp.maximum(c_ref[...][:, 0:1], 1.0)
        x = x / c
        n = jnp.sqrt(jnp.sum(x * x, axis=-1, keepdims=True))
        y = x / jnp.maximum(n, 1e-12)
        ro_ref[...] = r_ref[...] + y
        eo_ref[...] = y

    return pl.pallas_call(
        body,
        grid=(NE // R,),
        in_specs=[
            pl.BlockSpec((4, R, 32), lambda i: (0, i, 0)),
            pl.BlockSpec((R, 32), lambda i: (i, 0)),
            pl.BlockSpec((R, 128), lambda i: (i, 0)),
        ],
        out_specs=[pl.BlockSpec((R, 128), lambda i: (i, 0))] * 2,
        out_shape=[jax.ShapeDtypeStruct((NE, 128), jnp.float32)] * 2,
    )(aggq, cnt, res_in)


def _norm_user(aggq, res_in):
    R = 400

    def body(a_ref, r_ref, ro_ref):
        a = a_ref[...]
        x = jnp.concatenate([a[0], a[1], a[2], a[3]], axis=-1)
        n = jnp.sqrt(jnp.sum(x * x, axis=-1, keepdims=True))
        y = x / jnp.maximum(n, 1e-12)
        ro_ref[...] = r_ref[...] + y

    return pl.pallas_call(
        body,
        grid=(NU // R,),
        in_specs=[
            pl.BlockSpec((4, R, 32), lambda i: (0, i, 0)),
            pl.BlockSpec((R, 128), lambda i: (i, 0)),
        ],
        out_specs=pl.BlockSpec((R, 128), lambda i: (i, 0)),
        out_shape=jax.ShapeDtypeStruct((NU, 128), jnp.float32),
    )(aggq, res_in)


def _agg_pass(is_kg, q, sid, eqq, gsrc, ssrc, aux, wflat, zeros32, out, acc,
              tbuf, hbuf, ybuf, bufs, wv, gsems, ssems, isem):
    nsup = ET if is_kg else IT
    tot, stripe = (ACC_E, ES) if is_kg else (ACC_U, US)
    off = pl.multiple_of(jnp.minimum(sid * stripe, tot - stripe), 8)
    if is_kg:
        pltpu.sync_copy(wflat.at[q], wv)
        pltpu.sync_copy(zeros32, acc.at[pl.ds(off, stripe)])
    else:
        pltpu.sync_copy(zeros32.at[pl.ds(0, US)], acc.at[pl.ds(off, US)])
    plsc.subcore_barrier()

    def mul(buf, row):
        if is_kg:
            @plsc.parallel_loop(0, 128, step=16, unroll=2)
            def _m(i):
                tv = ybuf[row, pl.ds(i, 16)] - 1
                for l in range(16):
                    woff = tv[l] * QW
                    w0 = wv[pl.ds(woff, 16)]
                    w1 = wv[pl.ds(woff + 16, 16)]
                    buf[i + l, pl.ds(0, 16)] = buf[i + l, pl.ds(0, 16)] * w0
                    buf[i + l, pl.ds(16, 16)] = buf[i + l, pl.ds(16, 16)] * w1
        else:
            @plsc.parallel_loop(0, 128, step=16, unroll=2)
            def _m(i):
                vv = plsc.bitcast(ybuf[row, pl.ds(i, 16)], jnp.float32)
                for l in range(16):
                    v = vv[l]
                    buf[i + l, pl.ds(0, 16)] = buf[i + l, pl.ds(0, 16)] * v
                    buf[i + l, pl.ds(16, 16)] = buf[i + l, pl.ds(16, 16)] * v

    base = sid * nsup

    def g_issue(row, b):
        pltpu.async_copy(eqq.at[tbuf.at[row]], bufs[b], gsems[b])

    def g_wait(b):
        pltpu.make_async_copy(eqq.at[tbuf.at[0]], bufs[b], gsems[b]).wait()

    def s_issue(row, b):
        pltpu.async_copy(bufs[b], acc.at[hbuf.at[row]], ssems[b], add=True)

    def s_drain(b):
        pltpu.make_async_copy(bufs[b], acc.at[hbuf.at[0]], ssems[b]).wait()

    pltpu.sync_copy(gsrc.at[base], tbuf.at[pl.ds(0, SUP)])
    pltpu.sync_copy(ssrc.at[base], hbuf.at[pl.ds(0, SUP)])
    pltpu.sync_copy(aux.at[base], ybuf.at[pl.ds(0, SUP)])
    g_issue(0, 0)
    g_issue(1, 1)

    def super_body(s, carry):
        par = lax.rem(s, 3) * PAR
        npar = lax.rem(s + 1, 3) * PAR

        @pl.when(s < nsup - 1)
        def _():
            pltpu.async_copy(gsrc.at[base + s + 1],
                             tbuf.at[pl.ds(npar, SUP)], isem)
            pltpu.async_copy(ssrc.at[base + s + 1],
                             hbuf.at[pl.ds(npar, SUP)], isem)
            pltpu.async_copy(aux.at[base + s + 1],
                             ybuf.at[pl.ds(npar, SUP)], isem)

        for m in range(SUP):
            b = m % 4
            nb = (m + 2) % 4
            if m < 2:
                @pl.when(s > 0)
                def _():
                    s_drain(nb)
            else:
                s_drain(nb)
            if m < SUP - 2:
                g_issue(par + m + 2, nb)
            else:
                @pl.when(s < nsup - 1)
                def _():
                    if m == SUP - 2:
                        for aref in (tbuf, hbuf, ybuf):
                            pltpu.make_async_copy(
                                gsrc.at[base],
                                aref.at[pl.ds(npar, SUP)], isem).wait()
                    g_issue(npar + (m - (SUP - 2)), nb)
            g_wait(b)
            mul(bufs[b], par + m)
            s_issue(par + m, b)
        return carry

    lax.fori_loop(0, nsup, super_body, 0)
    s_drain((SUP * nsup - 2) % 4)
    s_drain((SUP * nsup - 1) % 4)
    plsc.subcore_barrier()
    pltpu.sync_copy(acc.at[pl.ds(off, stripe)],
                    out.at[q].at[pl.ds(off, stripe)])
    plsc.subcore_barrier()


def _counts_pass(sid, ssrc, ones32, zeros32, oc, acc, hbuf, X0):
    off = pl.multiple_of(jnp.minimum(sid * ES, ACC_E - ES), 8)
    pltpu.sync_copy(zeros32, acc.at[pl.ds(off, ES)])
    plsc.subcore_barrier()
    pltpu.sync_copy(ones32, X0)
    base = sid * ET

    def super_body(s, carry):
        pltpu.sync_copy(ssrc.at[base + s], hbuf.at[pl.ds(0, SUP)])
        for k in range(SUP):
            pltpu.sync_copy(X0, acc.at[hbuf.at[k]], add=True)
        return carry

    lax.fori_loop(0, ET, super_body, 0)
    plsc.subcore_barrier()
    pltpu.sync_copy(acc.at[pl.ds(off, ES)], oc.at[pl.ds(off, ES)])
    plsc.subcore_barrier()


def _sc_body(do_counts, eq, heads, tails, types, irows, icols, ivals,
             wflat, ones32, zeros32, oe, ou, oc,
             acc, tbuf, hbuf, ybuf, B0, B1, B2, B3, wv,
             g0, g1, g2, g3, s0, s1, s2, s3, isem):
    cid = lax.axis_index("c")
    sid = lax.axis_index("s")
    bufs = (B0, B1, B2, B3)
    gsems = (g0, g1, g2, g3)
    ssems = (s0, s1, s2, s3)

    if do_counts:
        @pl.when(cid == 0)
        def _():
            _counts_pass(sid, heads, ones32, zeros32, oc, acc, hbuf, B0)

    def q_body(qi, carry):
        q = cid * 2 + qi
        eqq = eq.at[q]
        _agg_pass(True, q, sid, eqq, tails, heads, types, wflat, zeros32,
                  oe, acc, tbuf, hbuf, ybuf, bufs, wv, gsems, ssems, isem)
        _agg_pass(False, q, sid, eqq, icols, irows, ivals, wflat, zeros32,
                  ou, acc, tbuf, hbuf, ybuf, bufs, wv, gsems, ssems, isem)
        return carry

    lax.fori_loop(0, 2, q_body, 0)


@functools.lru_cache(maxsize=2)
def _sc_agg(do_counts):
    mesh = plsc.VectorSubcoreMesh(core_axis_name="c", subcore_axis_name="s",
                                  num_cores=2, num_subcores=16)
    out_type = [
        jax.ShapeDtypeStruct((4, ACC_E, 32), jnp.float32),
        jax.ShapeDtypeStruct((4, ACC_U, 32), jnp.float32),
        jax.ShapeDtypeStruct((ACC_E, 32), jnp.float32),
    ]
    scratch = [
        pltpu.VMEM_SHARED((ACC_E, 32), jnp.float32),
        pltpu.VMEM((3 * PAR, 128), jnp.int32),
        pltpu.VMEM((3 * PAR, 128), jnp.int32),
        pltpu.VMEM((3 * PAR, 128), jnp.int32),
        pltpu.VMEM((128, 32), jnp.float32),
        pltpu.VMEM((128, 32), jnp.float32),
        pltpu.VMEM((128, 32), jnp.float32),
        pltpu.VMEM((128, 32), jnp.float32),
        pltpu.VMEM((384,), jnp.float32),
    ] + [pltpu.SemaphoreType.DMA] * 9
    return pl.kernel(
        functools.partial(_sc_body, do_counts),
        out_type=out_type,
        mesh=mesh,
        scratch_types=scratch,
        compiler_params=pltpu.CompilerParams(use_tc_tiling_on_sc=False,
                                             needs_layout_passes=False),
    )


def kernel(user_emb, entity_emb, edge_index, edge_type, interact_rows,
           interact_cols, interact_values, region_weight_mat, weight):
    head = edge_index[0]
    tail = edge_index[1]
    pe = EP - NEDGE
    heads_p = jnp.concatenate(
        [head, jnp.full((pe,), DUM_E, jnp.int32)]).reshape(ER, SUP, 128)
    tails_p = jnp.concatenate(
        [tail, jnp.zeros((pe,), jnp.int32)]).reshape(ER, SUP, 128)
    types_p = jnp.concatenate(
        [edge_type, jnp.ones((pe,), jnp.int32)]).reshape(ER, SUP, 128)
    pi = IP - NNZ
    irows_p = jnp.concatenate(
        [interact_rows,
         jnp.full((pi,), DUM_U, jnp.int32)]).reshape(IR, SUP, 128)
    icols_p = jnp.concatenate(
        [interact_cols, jnp.zeros((pi,), jnp.int32)]).reshape(IR, SUP, 128)
    ivals_p = lax.bitcast_convert_type(
        jnp.concatenate(
            [interact_values,
             jnp.zeros((pi,), jnp.float32)]).reshape(IR, SUP, 128),
        jnp.int32)
    wflat = jnp.pad(
        weight.reshape(11, 4, 32).transpose(1, 0, 2).reshape(4, 352),
        ((0, 0), (0, 32)))
    ones32 = jnp.ones((128, 32), jnp.float32)
    zeros32 = jnp.zeros((ES, 32), jnp.float32)
    rwm_pad = jnp.pad(region_weight_mat, ((0, RPAD - RD), (0, RPAD - RD)))

    ent = entity_emb
    e_res = entity_emb
    u_res = user_emb
    cnt = None
    for hop in range(2):
        ent_pad = jnp.pad(ent[LO:HI], ((0, RPAD - RD), (0, 0)))
        e_region = _region_blend(rwm_pad, ent_pad)[:RD]
        e = jnp.concatenate([ent[:LO], e_region, ent[HI:]], axis=0)
        eq = e.reshape(NE, 4, 32).transpose(1, 0, 2)
        oe, ou, oc = _sc_agg(hop == 0)(
            eq, heads_p, tails_p, types_p,
            irows_p, icols_p, ivals_p, wflat, ones32, zeros32)
        if hop == 0:
            cnt = oc[:NE]
        e_res, ent = _norm_ent(oe[:, :NE], cnt, e_res)
        u_res = _norm_user(ou[:, :NU], u_res)
    return e_res, u_res
